# Initial kernel scaffold; baseline (speedup 1.0000x reference)
import functools

import jax
import jax.numpy as jnp
from jax.experimental import pallas as pl
from jax.experimental.pallas import tpu as pltpu

B = 16
H = 32
W = 32
C = 768
HEADS = 8
HD = C // HEADS
WIN = 8
NH = H // WIN
NW = W // WIN
NR = NH * NW
W2 = WIN * WIN
TOPK = 4
KS = 5
M = B * NR * W2
SCALE = HD ** -0.5

_BM = 512


def _qkv_body(x_ref, w_ref, b_ref, q_ref, k_ref, v_ref):
    acc = jnp.dot(x_ref[...], w_ref[...], preferred_element_type=jnp.float32)
    acc = acc + b_ref[...]
    q_ref[...] = acc[:, :C]
    k_ref[...] = acc[:, C:2 * C]
    v_ref[...] = acc[:, 2 * C:]


def _qkv_call(xw, Wqkv, bqkv):
    grid = (M // _BM,)
    out = jax.ShapeDtypeStruct((M, C), jnp.float32)
    return pl.pallas_call(
        _qkv_body,
        grid=grid,
        in_specs=[
            pl.BlockSpec((_BM, C), lambda i: (i, 0)),
            pl.BlockSpec((C, 3 * C), lambda i: (0, 0)),
            pl.BlockSpec((1, 3 * C), lambda i: (0, 0)),
        ],
        out_specs=[
            pl.BlockSpec((_BM, C), lambda i: (i, 0)),
            pl.BlockSpec((_BM, C), lambda i: (i, 0)),
            pl.BlockSpec((_BM, C), lambda i: (i, 0)),
        ],
        out_shape=[out, out, out],
    )(xw, Wqkv, bqkv)


def _route_body(x_ref, w_ref, b_ref, r_ref):
    xm = jnp.mean(x_ref[0], axis=1)
    qkm = jnp.dot(xm, w_ref[...], preferred_element_type=jnp.float32)
    qkm = qkm + b_ref[...]
    qm = qkm[:, :C]
    km = qkm[:, C:]
    a = jax.lax.dot_general(qm, km, (((1,), (1,)), ((), ())),
                            preferred_element_type=jnp.float32)
    col = jax.lax.broadcasted_iota(jnp.int32, (NR, NR), 1)
    rows = []
    work = a
    for _ in range(TOPK):
        m = jnp.max(work, axis=1, keepdims=True)
        idx_t = jnp.min(jnp.where(work == m, col, NR), axis=1)
        work = jnp.where(col == idx_t[:, None], -1e30, work)
        rows.append(idx_t[None, :])
    rows.append(jnp.zeros((8 - TOPK, NR), jnp.int32))
    r_ref[0] = jnp.concatenate(rows, axis=0)


def _route_call(xw4, Wqk, bqk):
    return pl.pallas_call(
        _route_body,
        grid=(B,),
        in_specs=[
            pl.BlockSpec((1, NR, W2, C), lambda b: (b, 0, 0, 0)),
            pl.BlockSpec((C, 2 * C), lambda b: (0, 0)),
            pl.BlockSpec((1, 2 * C), lambda b: (0, 0)),
        ],
        out_specs=pl.BlockSpec((1, 8, NR), lambda b: (b, 0, 0)),
        out_shape=jax.ShapeDtypeStruct((B, 8, NR), jnp.int32),
    )(xw4, Wqk, bqk)


def _attn_body(r_ref, q_ref, k_ref, v_ref, o_ref):
    b = pl.program_id(0)
    r = pl.program_id(1)
    kt = []
    vt = []
    for t in range(TOPK):
        idx = r_ref[b, t, r]
        kt.append(k_ref[0, idx])
        vt.append(v_ref[0, idx])
    kg = jnp.concatenate(kt, axis=1)
    vg = jnp.concatenate(vt, axis=1)
    q = q_ref[0, 0] * SCALE
    s = jax.lax.dot_general(q, kg, (((2,), (2,)), ((0,), (0,))),
                            preferred_element_type=jnp.float32)
    s = s - jnp.max(s, axis=2, keepdims=True)
    p = jnp.exp(s)
    p = p / jnp.sum(p, axis=2, keepdims=True)
    o_ref[0, 0] = jax.lax.dot_general(p, vg, (((2,), (1,)), ((0,), (0,))),
                                      preferred_element_type=jnp.float32)


def _attn_call(r_idx, q5, k5, v5):
    blk_bat = pl.BlockSpec((1, NR, HEADS, W2, HD), lambda b, r: (b, 0, 0, 0, 0))
    blk_one = pl.BlockSpec((1, 1, HEADS, W2, HD), lambda b, r: (b, r, 0, 0, 0))
    return pl.pallas_call(
        _attn_body,
        grid=(B, NR),
        in_specs=[
            pl.BlockSpec(memory_space=pltpu.SMEM),
            blk_one,
            blk_bat,
            blk_bat,
        ],
        out_specs=blk_one,
        out_shape=jax.ShapeDtypeStruct((B, NR, HEADS, W2, HD), jnp.float32),
    )(r_idx, q5, k5, v5)


_HP = H + KS
_WP = W + KS - 1
_FP = _HP * _WP
_FV = H * _WP


def _lepe_body(x_ref, t_ref, b_ref, o_ref):
    acc = jnp.zeros((_FV, 6, 128), jnp.float32) + b_ref[0]
    for dy in range(KS):
        for dx in range(KS):
            s = dy * _WP + dx
            acc += x_ref[0, s:s + _FV] * t_ref[dy * KS + dx]
    o_ref[0] = acc.reshape(H, _WP, 6, 128)[:, :W]


def _lepe_call(vp, taps, lb):
    return pl.pallas_call(
        _lepe_body,
        grid=(B,),
        in_specs=[
            pl.BlockSpec((1, _FP, 6, 128), lambda b: (b, 0, 0, 0)),
            pl.BlockSpec((KS * KS, 6, 128), lambda b: (0, 0, 0)),
            pl.BlockSpec((1, 6, 128), lambda b: (0, 0, 0)),
        ],
        out_specs=pl.BlockSpec((1, H, W, 6, 128), lambda b: (b, 0, 0, 0, 0)),
        out_shape=jax.ShapeDtypeStruct((B, H, W, 6, 128), jnp.float32),
    )(vp, taps, lb)


_BE = 1024


def _proj_body(a_ref, l_ref, w_ref, b_ref, o_ref):
    acc = jnp.dot(a_ref[...] + l_ref[...], w_ref[...],
                  preferred_element_type=jnp.float32)
    o_ref[...] = acc + b_ref[...]


def _proj_call(a, lepe, Wo, bo):
    grid = (M // _BE,)
    return pl.pallas_call(
        _proj_body,
        grid=grid,
        in_specs=[
            pl.BlockSpec((_BE, C), lambda i: (i, 0)),
            pl.BlockSpec((_BE, C), lambda i: (i, 0)),
            pl.BlockSpec((C, C), lambda i: (0, 0)),
            pl.BlockSpec((1, C), lambda i: (0, 0)),
        ],
        out_specs=pl.BlockSpec((_BE, C), lambda i: (i, 0)),
        out_shape=jax.ShapeDtypeStruct((M, C), jnp.float32),
    )(a, lepe, Wo, bo)


@jax.jit
def _run(x, Wqkv, bqkv, Wo, bo, lepe_w, lepe_b):
    xw = (x.reshape(B, NH, WIN, NW, WIN, C)
          .transpose(0, 1, 3, 2, 4, 5)
          .reshape(M, C))
    q, k, v = _qkv_call(xw, Wqkv, bqkv.reshape(1, 3 * C))

    r_idx = _route_call(xw.reshape(B, NR, W2, C), Wqkv[:, :2 * C],
                        bqkv[:2 * C].reshape(1, 2 * C))

    def to5(t):
        return (t.reshape(B, NR, W2, HEADS, HD)
                .transpose(0, 1, 3, 2, 4))
    attn5 = _attn_call(r_idx, to5(q), to5(k), to5(v))

    v_img = (v.reshape(B, NH, NW, WIN, WIN, C)
             .transpose(0, 1, 3, 2, 4, 5)
             .reshape(B, H, W, C))
    vp = jnp.pad(v_img, ((0, 0), (2, 3), (2, 2), (0, 0)))
    vp = vp.reshape(B, _FP, 6, 128)
    taps = lepe_w.reshape(C, KS * KS).T.reshape(KS * KS, 6, 128)
    lepe = _lepe_call(vp, taps, lepe_b.reshape(1, 6, 128))
    lepe = lepe.reshape(B * H * W, C)

    a_tok = (attn5.reshape(B, NH, NW, HEADS, WIN, WIN, HD)
             .transpose(0, 1, 4, 2, 5, 3, 6)
             .reshape(B * H * W, C))

    out = _proj_call(a_tok, lepe, Wo, bo.reshape(1, C))
    return out.reshape(B, H * W, C)


def kernel(x, x_size, Wqkv, bqkv, Wo, bo, lepe_w, lepe_b):
    return _run(x, Wqkv, bqkv, Wo, bo, lepe_w, lepe_b)


# speedup vs baseline: 1.6442x; 1.6442x over previous
import jax
import jax.numpy as jnp
from jax.experimental import pallas as pl
from jax.experimental.pallas import tpu as pltpu

B = 16
H = 32
W = 32
C = 768
HEADS = 8
HD = C // HEADS
WIN = 8
NH = H // WIN
NW = W // WIN
NR = NH * NW
W2 = WIN * WIN
TOPK = 4
KS = 5
M = B * NR * W2
SCALE = HD ** -0.5

_BM = 512
_RB = _BM // W2


def _qkv_body(x_ref, w_ref, b_ref, q_ref, k_ref, v_ref, qm_ref, km_ref):
    acc = jnp.dot(x_ref[...], w_ref[...], preferred_element_type=jnp.float32)
    acc = acc + b_ref[...]
    qb = acc[:, :C]
    kb = acc[:, C:2 * C]
    q_ref[...] = qb.astype(jnp.bfloat16)
    k_ref[...] = kb.astype(jnp.bfloat16)
    v_ref[...] = acc[:, 2 * C:].astype(jnp.bfloat16)
    qm_ref[...] = jnp.mean(qb.reshape(_RB, W2, C), axis=1)
    km_ref[...] = jnp.mean(kb.reshape(_RB, W2, C), axis=1)


def _qkv_call(xw16, Wqkv16, bqkv):
    grid = (M // _BM,)
    out = jax.ShapeDtypeStruct((M, C), jnp.bfloat16)
    outm = jax.ShapeDtypeStruct((B * NR, C), jnp.float32)
    return pl.pallas_call(
        _qkv_body,
        grid=grid,
        in_specs=[
            pl.BlockSpec((_BM, C), lambda i: (i, 0)),
            pl.BlockSpec((C, 3 * C), lambda i: (0, 0)),
            pl.BlockSpec((1, 3 * C), lambda i: (0, 0)),
        ],
        out_specs=[
            pl.BlockSpec((_BM, C), lambda i: (i, 0)),
            pl.BlockSpec((_BM, C), lambda i: (i, 0)),
            pl.BlockSpec((_BM, C), lambda i: (i, 0)),
            pl.BlockSpec((_RB, C), lambda i: (i, 0)),
            pl.BlockSpec((_RB, C), lambda i: (i, 0)),
        ],
        out_shape=[out, out, out, outm, outm],
    )(xw16, Wqkv16, bqkv)


def _route_body(qm_ref, km_ref, r_ref):
    qm = qm_ref[0].astype(jnp.bfloat16)
    km = km_ref[0].astype(jnp.bfloat16)
    a = jax.lax.dot_general(qm, km, (((1,), (1,)), ((), ())),
                            preferred_element_type=jnp.float32)
    col = jax.lax.broadcasted_iota(jnp.int32, (NR, NR), 1)
    rows = []
    work = a
    for _ in range(TOPK):
        m = jnp.max(work, axis=1, keepdims=True)
        idx_t = jnp.min(jnp.where(work == m, col, NR), axis=1)
        work = jnp.where(col == idx_t[:, None], -1e30, work)
        rows.append(idx_t[None, :])
    rows.append(jnp.zeros((8 - TOPK, NR), jnp.int32))
    r_ref[0] = jnp.concatenate(rows, axis=0)


def _route_call(qm3, km3):
    return pl.pallas_call(
        _route_body,
        grid=(B,),
        in_specs=[
            pl.BlockSpec((1, NR, C), lambda b: (b, 0, 0)),
            pl.BlockSpec((1, NR, C), lambda b: (b, 0, 0)),
        ],
        out_specs=pl.BlockSpec((1, 8, NR), lambda b: (b, 0, 0)),
        out_shape=jax.ShapeDtypeStruct((B, 8, NR), jnp.int32),
    )(qm3, km3)


def _attn_body(r_ref, q_ref, k_ref, v_ref, o_ref):
    b = pl.program_id(0)
    r = pl.program_id(1)
    kt = []
    vt = []
    for t in range(TOPK):
        idx = r_ref[b, t, r]
        kt.append(k_ref[0, idx])
        vt.append(v_ref[0, idx])
    kg = jnp.concatenate(kt, axis=1)
    vg = jnp.concatenate(vt, axis=1)
    q = q_ref[0, 0]
    s = jax.lax.dot_general(q, kg, (((2,), (2,)), ((0,), (0,))),
                            preferred_element_type=jnp.float32)
    s = s * SCALE
    s = s - jnp.max(s, axis=2, keepdims=True)
    p = jnp.exp(s)
    p = p / jnp.sum(p, axis=2, keepdims=True)
    o_ref[0, 0] = jax.lax.dot_general(
        p.astype(jnp.bfloat16), vg, (((2,), (1,)), ((0,), (0,))),
        preferred_element_type=jnp.float32).astype(jnp.bfloat16)


def _attn_call(r_idx, q5, k5, v5):
    blk_bat = pl.BlockSpec((1, NR, HEADS, W2, HD), lambda b, r: (b, 0, 0, 0, 0))
    blk_one = pl.BlockSpec((1, 1, HEADS, W2, HD), lambda b, r: (b, r, 0, 0, 0))
    return pl.pallas_call(
        _attn_body,
        grid=(B, NR),
        in_specs=[
            pl.BlockSpec(memory_space=pltpu.SMEM),
            blk_one,
            blk_bat,
            blk_bat,
        ],
        out_specs=blk_one,
        out_shape=jax.ShapeDtypeStruct((B, NR, HEADS, W2, HD), jnp.bfloat16),
    )(r_idx, q5, k5, v5)


_HP = H + KS
_WP = W + KS - 1
_FP = _HP * _WP
_FV = H * _WP


def _lepe_body(x_ref, t_ref, b_ref, o_ref):
    acc = jnp.zeros((_FV, 6, 128), jnp.float32) + b_ref[0]
    for dy in range(KS):
        for dx in range(KS):
            s = dy * _WP + dx
            acc += x_ref[0, s:s + _FV].astype(jnp.float32) * t_ref[dy * KS + dx]
    o_ref[0] = acc.reshape(H, _WP, 6, 128)[:, :W].astype(jnp.bfloat16)


def _lepe_call(vp, taps, lb):
    return pl.pallas_call(
        _lepe_body,
        grid=(B,),
        in_specs=[
            pl.BlockSpec((1, _FP, 6, 128), lambda b: (b, 0, 0, 0)),
            pl.BlockSpec((KS * KS, 6, 128), lambda b: (0, 0, 0)),
            pl.BlockSpec((1, 6, 128), lambda b: (0, 0, 0)),
        ],
        out_specs=pl.BlockSpec((1, H, W, 6, 128), lambda b: (b, 0, 0, 0, 0)),
        out_shape=jax.ShapeDtypeStruct((B, H, W, 6, 128), jnp.bfloat16),
    )(vp, taps, lb)


_BE = 1024


def _proj_body(a_ref, l_ref, w_ref, b_ref, o_ref):
    s = (a_ref[...].astype(jnp.float32)
         + l_ref[...].astype(jnp.float32)).astype(jnp.bfloat16)
    acc = jnp.dot(s, w_ref[...], preferred_element_type=jnp.float32)
    o_ref[...] = acc + b_ref[...]


def _proj_call(a, lepe, Wo16, bo):
    grid = (M // _BE,)
    return pl.pallas_call(
        _proj_body,
        grid=grid,
        in_specs=[
            pl.BlockSpec((_BE, C), lambda i: (i, 0)),
            pl.BlockSpec((_BE, C), lambda i: (i, 0)),
            pl.BlockSpec((C, C), lambda i: (0, 0)),
            pl.BlockSpec((1, C), lambda i: (0, 0)),
        ],
        out_specs=pl.BlockSpec((_BE, C), lambda i: (i, 0)),
        out_shape=jax.ShapeDtypeStruct((M, C), jnp.float32),
    )(a, lepe, Wo16, bo)


@jax.jit
def _run(x, Wqkv, bqkv, Wo, bo, lepe_w, lepe_b):
    xw = (x.reshape(B, NH, WIN, NW, WIN, C)
          .transpose(0, 1, 3, 2, 4, 5)
          .reshape(M, C))
    q, k, v, qm, km = _qkv_call(xw.astype(jnp.bfloat16),
                                Wqkv.astype(jnp.bfloat16),
                                bqkv.reshape(1, 3 * C))

    r_idx = _route_call(qm.reshape(B, NR, C), km.reshape(B, NR, C))

    def to5(t):
        return (t.reshape(B, NR, W2, HEADS, HD)
                .transpose(0, 1, 3, 2, 4))
    attn5 = _attn_call(r_idx, to5(q), to5(k), to5(v))

    v_img = (v.reshape(B, NH, NW, WIN, WIN, C)
             .transpose(0, 1, 3, 2, 4, 5)
             .reshape(B, H, W, C))
    vp = jnp.pad(v_img, ((0, 0), (2, 3), (2, 2), (0, 0)))
    vp = vp.reshape(B, _FP, 6, 128)
    taps = lepe_w.reshape(C, KS * KS).T.reshape(KS * KS, 6, 128)
    lepe = _lepe_call(vp, taps, lepe_b.reshape(1, 6, 128))
    lepe = lepe.reshape(B * H * W, C)

    a_tok = (attn5.reshape(B, NH, NW, HEADS, WIN, WIN, HD)
             .transpose(0, 1, 4, 2, 5, 3, 6)
             .reshape(B * H * W, C))

    out = _proj_call(a_tok, lepe, Wo.astype(jnp.bfloat16), bo.reshape(1, C))
    return out.reshape(B, H * W, C)


def kernel(x, x_size, Wqkv, bqkv, Wo, bo, lepe_w, lepe_b):
    return _run(x, Wqkv, bqkv, Wo, bo, lepe_w, lepe_b)


# speedup vs baseline: 3.5582x; 2.1640x over previous
import jax
import jax.numpy as jnp
from jax.experimental import pallas as pl
from jax.experimental.pallas import tpu as pltpu

B = 16
H = 32
W = 32
C = 768
HEADS = 8
HD = C // HEADS
HDP = 128
CP = HEADS * HDP
WIN = 8
NH = H // WIN
NW = W // WIN
NR = NH * NW
W2 = WIN * WIN
TOPK = 4
KS = 5
SCALE = HD ** -0.5


def _qkv_body(x_ref, w_ref, b_ref, q_ref, k_ref, v_ref, r_ref):
    x16 = x_ref[0].astype(jnp.bfloat16)
    parts = []
    for i, o_ref in enumerate((q_ref, k_ref, v_ref)):
        sl = slice(i * CP, (i + 1) * CP)
        p = jnp.dot(x16, w_ref[:, sl], preferred_element_type=jnp.float32)
        p = p + b_ref[:, sl]
        o_ref[0] = p.astype(jnp.bfloat16)
        parts.append(p)
    qf, kf = parts[0], parts[1]

    def rmean(t):
        t6 = t.reshape(NH, WIN, NW, WIN, CP)
        return t6.sum(axis=3).sum(axis=1).reshape(NR, CP) * (1.0 / W2)

    qm = rmean(qf).astype(jnp.bfloat16)
    km = rmean(kf).astype(jnp.bfloat16)
    a = jax.lax.dot_general(qm, km, (((1,), (1,)), ((), ())),
                            preferred_element_type=jnp.float32)
    col = jax.lax.broadcasted_iota(jnp.int32, (NR, NR), 1)
    rows = []
    work = a
    for _ in range(TOPK):
        mx = jnp.max(work, axis=1, keepdims=True)
        idx_t = jnp.min(jnp.where(work == mx, col, NR), axis=1)
        work = jnp.where(col == idx_t[:, None], -1e30, work)
        rows.append(idx_t[None, :])
    rows.append(jnp.zeros((8 - TOPK, NR), jnp.int32))
    r_ref[0] = jnp.concatenate(rows, axis=0)


def _qkv_call(x3, Wp16, bp):
    out = jax.ShapeDtypeStruct((B, H * W, CP), jnp.bfloat16)
    return pl.pallas_call(
        _qkv_body,
        grid=(B,),
        in_specs=[
            pl.BlockSpec((1, H * W, C), lambda b: (b, 0, 0)),
            pl.BlockSpec((C, 3 * CP), lambda b: (0, 0)),
            pl.BlockSpec((1, 3 * CP), lambda b: (0, 0)),
        ],
        out_specs=[
            pl.BlockSpec((1, H * W, CP), lambda b: (b, 0, 0)),
            pl.BlockSpec((1, H * W, CP), lambda b: (b, 0, 0)),
            pl.BlockSpec((1, H * W, CP), lambda b: (b, 0, 0)),
            pl.BlockSpec((1, 8, NR), lambda b: (b, 0, 0)),
        ],
        out_shape=[out, out, out,
                   jax.ShapeDtypeStruct((B, 8, NR), jnp.int32)],
    )(x3, Wp16, bp)


_RG = 8
_NG = NR // _RG
_GR = _RG // NW


def _attn_body(r_ref, q_ref, k_ref, v_ref, o_ref):
    b = pl.program_id(0)
    g = pl.program_id(1)
    qs, kgs, vgs = [], [], []
    for ii in range(_GR):
        for j in range(NW):
            r = _RG * g + ii * NW + j
            q = q_ref[0, ii, :, j].reshape(W2, CP)
            kts = []
            vts = []
            for t in range(TOPK):
                ri = r_ref[b, t, r]
                di, dj = ri // NW, ri % NW
                kts.append(k_ref[0, di, :, dj].reshape(W2, CP))
                vts.append(v_ref[0, di, :, dj].reshape(W2, CP))
            qs.append(q)
            kgs.append(jnp.concatenate(kts, axis=0))
            vgs.append(jnp.concatenate(vts, axis=0))
    ss = []
    for i in range(_RG):
        for h in range(HEADS):
            sl = slice(h * HDP, (h + 1) * HDP)
            ss.append(jax.lax.dot_general(
                qs[i][:, sl], kgs[i][:, sl], (((1,), (1,)), ((), ())),
                preferred_element_type=jnp.float32))
    ps = []
    for s in ss:
        s = s * SCALE
        s = s - jnp.max(s, axis=1, keepdims=True)
        p = jnp.exp(s)
        p = p / jnp.sum(p, axis=1, keepdims=True)
        ps.append(p.astype(jnp.bfloat16))
    for i in range(_RG):
        outs = []
        for h in range(HEADS):
            sl = slice(h * HDP, (h + 1) * HDP)
            outs.append(jnp.dot(ps[i * HEADS + h], vgs[i][:, sl],
                                preferred_element_type=jnp.float32))
        out = jnp.concatenate(outs, axis=1).astype(jnp.bfloat16)
        ii, j = i // NW, i % NW
        o_ref[0, ii, :, j] = out.reshape(WIN, WIN, CP)


def _attn_call(r_idx, q6, k6, v6):
    qblk = (1, _GR, WIN, NW, WIN, CP)

    def q_map(b, g, rr):
        return (b, g, 0, 0, 0, 0)

    def kv_map(b, g, rr):
        return (b, 0, 0, 0, 0, 0)

    grid_spec = pltpu.PrefetchScalarGridSpec(
        num_scalar_prefetch=1,
        grid=(B, _NG),
        in_specs=[
            pl.BlockSpec(qblk, q_map),
            pl.BlockSpec((1, NH, WIN, NW, WIN, CP), kv_map),
            pl.BlockSpec((1, NH, WIN, NW, WIN, CP), kv_map),
        ],
        out_specs=pl.BlockSpec(qblk, q_map),
    )
    return pl.pallas_call(
        _attn_body,
        grid_spec=grid_spec,
        out_shape=jax.ShapeDtypeStruct((B, NH, WIN, NW, WIN, CP),
                                       jnp.bfloat16),
    )(r_idx, q6, k6, v6)


_HP = H + KS
_WP = W + KS - 1
_FP = _HP * _WP
_FV = H * _WP


def _proj_body(a_ref, v_ref, t_ref, lb_ref, w_ref, bo_ref, o_ref, pad_ref):
    pad_ref[...] = jnp.zeros((_FP, HEADS, HDP), jnp.bfloat16)
    for y in range(H):
        base = (y + 2) * _WP + 2
        pad_ref[base:base + W] = v_ref[0, y * W:(y + 1) * W]
    xs = pad_ref[...]
    acc = jnp.zeros((_FV, HEADS, HDP), jnp.bfloat16) + lb_ref[0]
    for dy in range(KS):
        for dx in range(KS):
            s = dy * _WP + dx
            acc += xs[s:s + _FV] * t_ref[dy * KS + dx]
    lepe = acc.reshape(H, _WP, HEADS, HDP)[:, :W].reshape(H * W, HEADS, HDP)
    ssum = (a_ref[0] + lepe).reshape(H * W, CP)
    o_ref[0] = jnp.dot(ssum, w_ref[...],
                       preferred_element_type=jnp.float32) + bo_ref[...]


def _proj_call(a4, v4, taps, lb, Wop16, bo):
    return pl.pallas_call(
        _proj_body,
        grid=(B,),
        in_specs=[
            pl.BlockSpec((1, H * W, HEADS, HDP), lambda b: (b, 0, 0, 0)),
            pl.BlockSpec((1, H * W, HEADS, HDP), lambda b: (b, 0, 0, 0)),
            pl.BlockSpec((KS * KS, HEADS, HDP), lambda b: (0, 0, 0)),
            pl.BlockSpec((1, HEADS, HDP), lambda b: (0, 0, 0)),
            pl.BlockSpec((CP, C), lambda b: (0, 0)),
            pl.BlockSpec((1, C), lambda b: (0, 0)),
        ],
        out_specs=pl.BlockSpec((1, H * W, C), lambda b: (b, 0, 0)),
        out_shape=jax.ShapeDtypeStruct((B, H * W, C), jnp.float32),
        scratch_shapes=[pltpu.VMEM((_FP, HEADS, HDP), jnp.bfloat16)],
    )(a4, v4, taps, lb, Wop16, bo)


def _pad_heads(t):
    lead = t.shape[:-1]
    n = t.shape[-1] // HD
    t = t.reshape(lead + (n, HD))
    t = jnp.pad(t, [(0, 0)] * len(lead) + [(0, 0), (0, HDP - HD)])
    return t.reshape(lead + (n * HDP,))


@jax.jit
def _run(x, Wqkv, bqkv, Wo, bo, lepe_w, lepe_b):
    Wp16 = _pad_heads(Wqkv).astype(jnp.bfloat16)
    bp = _pad_heads(bqkv).reshape(1, 3 * CP)
    Wop16 = (jnp.pad(Wo.reshape(HEADS, HD, C),
                     ((0, 0), (0, HDP - HD), (0, 0)))
             .reshape(CP, C).astype(jnp.bfloat16))
    taps = _pad_heads(lepe_w.reshape(C, KS * KS).T).reshape(KS * KS, HEADS, HDP)
    lb = _pad_heads(lepe_b).reshape(1, HEADS, HDP)

    q, k, v, r_idx = _qkv_call(x, Wp16, bp)
    shp6 = (B, NH, WIN, NW, WIN, CP)
    attn = _attn_call(r_idx, q.reshape(shp6), k.reshape(shp6),
                      v.reshape(shp6))
    out = _proj_call(attn.reshape(B, H * W, HEADS, HDP),
                     v.reshape(B, H * W, HEADS, HDP),
                     taps.astype(jnp.bfloat16), lb.astype(jnp.bfloat16),
                     Wop16, bo.reshape(1, C))
    return out


def kernel(x, x_size, Wqkv, bqkv, Wo, bo, lepe_w, lepe_b):
    return _run(x, Wqkv, bqkv, Wo, bo, lepe_w, lepe_b)


# speedup vs baseline: 3.6022x; 1.0124x over previous
import jax
import jax.numpy as jnp
from jax.experimental import pallas as pl
from jax.experimental.pallas import tpu as pltpu

B = 16
H = 32
W = 32
C = 768
HEADS = 8
HD = C // HEADS
HDP = 128
CP = HEADS * HDP
WIN = 8
NH = H // WIN
NW = W // WIN
NR = NH * NW
W2 = WIN * WIN
TOPK = 4
KS = 5
SCALE = HD ** -0.5


def _qkv_body(x_ref, w_ref, b_ref, q_ref, k_ref, v_ref, r_ref):
    x16 = x_ref[0].astype(jnp.bfloat16)
    parts = []
    for i, o_ref in enumerate((q_ref, k_ref, v_ref)):
        sl = slice(i * CP, (i + 1) * CP)
        p = jnp.dot(x16, w_ref[:, sl], preferred_element_type=jnp.float32)
        p = p + b_ref[:, sl]
        o_ref[0] = p.astype(jnp.bfloat16)
        parts.append(p)
    qf, kf = parts[0], parts[1]

    def rmean(t):
        t6 = t.reshape(NH, WIN, NW, WIN, CP)
        return t6.sum(axis=3).sum(axis=1).reshape(NR, CP) * (1.0 / W2)

    qm = rmean(qf).astype(jnp.bfloat16)
    km = rmean(kf).astype(jnp.bfloat16)
    a = jax.lax.dot_general(qm, km, (((1,), (1,)), ((), ())),
                            preferred_element_type=jnp.float32)
    col = jax.lax.broadcasted_iota(jnp.int32, (NR, NR), 1)
    rows = []
    work = a
    for _ in range(TOPK):
        mx = jnp.max(work, axis=1, keepdims=True)
        idx_t = jnp.min(jnp.where(work == mx, col, NR), axis=1)
        work = jnp.where(col == idx_t[:, None], -1e30, work)
        rows.append(idx_t[None, :])
    rows.append(jnp.zeros((8 - TOPK, NR), jnp.int32))
    r_ref[0] = jnp.concatenate(rows, axis=0)


def _qkv_call(x3, Wp16, bp):
    out = jax.ShapeDtypeStruct((B, H * W, CP), jnp.bfloat16)
    return pl.pallas_call(
        _qkv_body,
        grid=(B,),
        in_specs=[
            pl.BlockSpec((1, H * W, C), lambda b: (b, 0, 0)),
            pl.BlockSpec((C, 3 * CP), lambda b: (0, 0)),
            pl.BlockSpec((1, 3 * CP), lambda b: (0, 0)),
        ],
        out_specs=[
            pl.BlockSpec((1, H * W, CP), lambda b: (b, 0, 0)),
            pl.BlockSpec((1, H * W, CP), lambda b: (b, 0, 0)),
            pl.BlockSpec((1, H * W, CP), lambda b: (b, 0, 0)),
            pl.BlockSpec((1, 8, NR), lambda b: (b, 0, 0)),
        ],
        out_shape=[out, out, out,
                   jax.ShapeDtypeStruct((B, 8, NR), jnp.int32)],
    )(x3, Wp16, bp)


_RG = 8
_NG = NR // _RG
_GR = _RG // NW


def _attn_body(r_ref, q_ref, k_ref, v_ref, o_ref):
    b = pl.program_id(0)
    g = pl.program_id(1)
    qs, kgs, vgs = [], [], []
    for ii in range(_GR):
        for j in range(NW):
            r = _RG * g + ii * NW + j
            q = q_ref[0, ii, :, j].reshape(W2, CP)
            kts = []
            vts = []
            for t in range(TOPK):
                ri = r_ref[b, t, r]
                di, dj = ri // NW, ri % NW
                kts.append(k_ref[0, di, :, dj].reshape(W2, CP))
                vts.append(v_ref[0, di, :, dj].reshape(W2, CP))
            qs.append(q)
            kgs.append(jnp.concatenate(kts, axis=0))
            vgs.append(jnp.concatenate(vts, axis=0))
    ss = []
    for i in range(_RG):
        for h in range(HEADS):
            sl = slice(h * HDP, (h + 1) * HDP)
            ss.append(jax.lax.dot_general(
                qs[i][:, sl], kgs[i][:, sl], (((1,), (1,)), ((), ())),
                preferred_element_type=jnp.float32))
    ps = []
    for s in ss:
        s = s * SCALE
        s = s - jnp.max(s, axis=1, keepdims=True)
        p = jnp.exp(s)
        p = p / jnp.sum(p, axis=1, keepdims=True)
        ps.append(p.astype(jnp.bfloat16))
    for i in range(_RG):
        outs = []
        for h in range(HEADS):
            sl = slice(h * HDP, (h + 1) * HDP)
            outs.append(jnp.dot(ps[i * HEADS + h], vgs[i][:, sl],
                                preferred_element_type=jnp.float32))
        out = jnp.concatenate(outs, axis=1).astype(jnp.bfloat16)
        ii, j = i // NW, i % NW
        o_ref[0, ii, :, j] = out.reshape(WIN, WIN, CP)


def _attn_call(r_idx, q6, k6, v6):
    qblk = (1, _GR, WIN, NW, WIN, CP)

    def q_map(b, g, rr):
        return (b, g, 0, 0, 0, 0)

    def kv_map(b, g, rr):
        return (b, 0, 0, 0, 0, 0)

    grid_spec = pltpu.PrefetchScalarGridSpec(
        num_scalar_prefetch=1,
        grid=(B, _NG),
        in_specs=[
            pl.BlockSpec(qblk, q_map),
            pl.BlockSpec((1, NH, WIN, NW, WIN, CP), kv_map),
            pl.BlockSpec((1, NH, WIN, NW, WIN, CP), kv_map),
        ],
        out_specs=pl.BlockSpec(qblk, q_map),
    )
    return pl.pallas_call(
        _attn_body,
        grid_spec=grid_spec,
        out_shape=jax.ShapeDtypeStruct((B, NH, WIN, NW, WIN, CP),
                                       jnp.bfloat16),
    )(r_idx, q6, k6, v6)


_HP = H + KS
_WP = W + KS - 1
_FP = _HP * _WP
_FV = H * _WP


def _proj_body(a_ref, v_ref, t_ref, lb_ref, w_ref, bo_ref, o_ref, pad_ref):
    pad_ref[...] = jnp.zeros((_FP, HEADS, HDP), jnp.bfloat16)
    for y in range(H):
        base = (y + 2) * _WP + 2
        pad_ref[base:base + W] = v_ref[0, y * W:(y + 1) * W]
    xs = pad_ref[...]
    acc = jnp.zeros((_FV, HEADS, HDP), jnp.float32) + lb_ref[0]
    for dy in range(KS):
        for dx in range(KS):
            s = dy * _WP + dx
            acc += xs[s:s + _FV].astype(jnp.float32) * t_ref[dy * KS + dx]
    lepe = acc.reshape(H, _WP, HEADS, HDP)[:, :W].reshape(H * W, HEADS, HDP)
    ssum = (a_ref[0].astype(jnp.float32) + lepe).astype(jnp.bfloat16).reshape(H * W, CP)
    o_ref[0] = jnp.dot(ssum, w_ref[...],
                       preferred_element_type=jnp.float32) + bo_ref[...]


def _proj_call(a4, v4, taps, lb, Wop16, bo):
    return pl.pallas_call(
        _proj_body,
        grid=(B,),
        in_specs=[
            pl.BlockSpec((1, H * W, HEADS, HDP), lambda b: (b, 0, 0, 0)),
            pl.BlockSpec((1, H * W, HEADS, HDP), lambda b: (b, 0, 0, 0)),
            pl.BlockSpec((KS * KS, HEADS, HDP), lambda b: (0, 0, 0)),
            pl.BlockSpec((1, HEADS, HDP), lambda b: (0, 0, 0)),
            pl.BlockSpec((CP, C), lambda b: (0, 0)),
            pl.BlockSpec((1, C), lambda b: (0, 0)),
        ],
        out_specs=pl.BlockSpec((1, H * W, C), lambda b: (b, 0, 0)),
        out_shape=jax.ShapeDtypeStruct((B, H * W, C), jnp.float32),
        scratch_shapes=[pltpu.VMEM((_FP, HEADS, HDP), jnp.bfloat16)],
    )(a4, v4, taps, lb, Wop16, bo)


def _pad_heads(t):
    lead = t.shape[:-1]
    n = t.shape[-1] // HD
    t = t.reshape(lead + (n, HD))
    t = jnp.pad(t, [(0, 0)] * len(lead) + [(0, 0), (0, HDP - HD)])
    return t.reshape(lead + (n * HDP,))


@jax.jit
def _run(x, Wqkv, bqkv, Wo, bo, lepe_w, lepe_b):
    Wp16 = _pad_heads(Wqkv).astype(jnp.bfloat16)
    bp = _pad_heads(bqkv).reshape(1, 3 * CP)
    Wop16 = (jnp.pad(Wo.reshape(HEADS, HD, C),
                     ((0, 0), (0, HDP - HD), (0, 0)))
             .reshape(CP, C).astype(jnp.bfloat16))
    taps = _pad_heads(lepe_w.reshape(C, KS * KS).T).reshape(KS * KS, HEADS, HDP)
    lb = _pad_heads(lepe_b).reshape(1, HEADS, HDP)

    q, k, v, r_idx = _qkv_call(x, Wp16, bp)
    shp6 = (B, NH, WIN, NW, WIN, CP)
    attn = _attn_call(r_idx, q.reshape(shp6), k.reshape(shp6),
                      v.reshape(shp6))
    out = _proj_call(attn.reshape(B, H * W, HEADS, HDP),
                     v.reshape(B, H * W, HEADS, HDP),
                     taps, lb, Wop16, bo.reshape(1, C))
    return out


def kernel(x, x_size, Wqkv, bqkv, Wo, bo, lepe_w, lepe_b):
    return _run(x, Wqkv, bqkv, Wo, bo, lepe_w, lepe_b)


# speedup vs baseline: 4.0143x; 1.1144x over previous
import jax
import jax.numpy as jnp
from jax.experimental import pallas as pl
from jax.experimental.pallas import tpu as pltpu

B = 16
H = 32
W = 32
C = 768
HEADS = 8
HD = C // HEADS
HDP = 128
CP = HEADS * HDP
WIN = 8
NH = H // WIN
NW = W // WIN
NR = NH * NW
W2 = WIN * WIN
TOPK = 4
KS = 5
SCALE = HD ** -0.5


def _qkv_body(x_ref, w_ref, b_ref, q_ref, k_ref, v_ref, r_ref):
    x16 = x_ref[0].astype(jnp.bfloat16)
    parts = []
    for i, o_ref in enumerate((q_ref, k_ref, v_ref)):
        sl = slice(i * CP, (i + 1) * CP)
        p = jnp.dot(x16, w_ref[:, sl], preferred_element_type=jnp.float32)
        p = p + b_ref[:, sl]
        o_ref[0] = p.astype(jnp.bfloat16)
        parts.append(p)
    qf, kf = parts[0], parts[1]

    def rmean(t):
        t6 = t.reshape(NH, WIN, NW, WIN, CP)
        return t6.sum(axis=3).sum(axis=1).reshape(NR, CP) * (1.0 / W2)

    qm = rmean(qf).astype(jnp.bfloat16)
    km = rmean(kf).astype(jnp.bfloat16)
    a = jax.lax.dot_general(qm, km, (((1,), (1,)), ((), ())),
                            preferred_element_type=jnp.float32)
    col = jax.lax.broadcasted_iota(jnp.int32, (NR, NR), 1)
    rows = []
    work = a
    for _ in range(TOPK):
        mx = jnp.max(work, axis=1, keepdims=True)
        idx_t = jnp.min(jnp.where(work == mx, col, NR), axis=1)
        work = jnp.where(col == idx_t[:, None], -1e30, work)
        rows.append(idx_t[None, :])
    rows.append(jnp.zeros((8 - TOPK, NR), jnp.int32))
    r_ref[0] = jnp.concatenate(rows, axis=0)


def _qkv_call(x3, Wp16, bp):
    out = jax.ShapeDtypeStruct((B, H * W, CP), jnp.bfloat16)
    return pl.pallas_call(
        _qkv_body,
        grid=(B,),
        in_specs=[
            pl.BlockSpec((1, H * W, C), lambda b: (b, 0, 0)),
            pl.BlockSpec((C, 3 * CP), lambda b: (0, 0)),
            pl.BlockSpec((1, 3 * CP), lambda b: (0, 0)),
        ],
        out_specs=[
            pl.BlockSpec((1, H * W, CP), lambda b: (b, 0, 0)),
            pl.BlockSpec((1, H * W, CP), lambda b: (b, 0, 0)),
            pl.BlockSpec((1, H * W, CP), lambda b: (b, 0, 0)),
            pl.BlockSpec((1, 8, NR), lambda b: (b, 0, 0)),
        ],
        out_shape=[out, out, out,
                   jax.ShapeDtypeStruct((B, 8, NR), jnp.int32)],
    )(x3, Wp16, bp)


_HP = H + KS
_WP = W + KS - 1
_FP = _HP * _WP
_FV = H * _WP
_CH = 8


def _fuse_body(r_ref, q_ref, k_ref, v_ref, t_ref, lb_ref, w_ref, bo_ref,
               o_ref, pad_ref):
    b = pl.program_id(0)
    att = []
    for c0 in range(0, NR, _CH):
        qs, kgs, vgs = [], [], []
        for r in range(c0, c0 + _CH):
            ii, j = r // NW, r % NW
            qs.append(q_ref[0, ii, :, j].reshape(W2, CP))
            kts, vts = [], []
            for t in range(TOPK):
                ri = r_ref[b, t, r]
                di, dj = ri // NW, ri % NW
                kts.append(k_ref[0, di, :, dj].reshape(W2, CP))
                vts.append(v_ref[0, di, :, dj].reshape(W2, CP))
            kgs.append(jnp.concatenate(kts, axis=0))
            vgs.append(jnp.concatenate(vts, axis=0))
        ss = []
        for i in range(_CH):
            for h in range(HEADS):
                sl = slice(h * HDP, (h + 1) * HDP)
                ss.append(jax.lax.dot_general(
                    qs[i][:, sl], kgs[i][:, sl], (((1,), (1,)), ((), ())),
                    preferred_element_type=jnp.float32))
        ps = []
        for s in ss:
            s = s * SCALE
            s = s - jnp.max(s, axis=1, keepdims=True)
            p = jnp.exp(s)
            p = p / jnp.sum(p, axis=1, keepdims=True)
            ps.append(p.astype(jnp.bfloat16))
        for i in range(_CH):
            outs = []
            for h in range(HEADS):
                sl = slice(h * HDP, (h + 1) * HDP)
                outs.append(jnp.dot(ps[i * HEADS + h], vgs[i][:, sl],
                                    preferred_element_type=jnp.float32))
            att.append(jnp.concatenate(outs, axis=1).astype(jnp.bfloat16))
    pad_ref[...] = jnp.zeros((_FP, HEADS, HDP), jnp.bfloat16)
    v4 = v_ref[0].reshape(H * W, HEADS, HDP)
    for y in range(H):
        base = (y + 2) * _WP + 2
        pad_ref[base:base + W] = v4[y * W:(y + 1) * W]
    xs = pad_ref[...]
    acc = jnp.zeros((_FV, HEADS, HDP), jnp.float32) + lb_ref[0]
    for dy in range(KS):
        for dx in range(KS):
            s = dy * _WP + dx
            acc += xs[s:s + _FV].astype(jnp.float32) * t_ref[dy * KS + dx]
    lepe = (acc.reshape(H, _WP, HEADS, HDP)[:, :W]
            .reshape(H * W, CP).astype(jnp.bfloat16))
    a_img = jnp.concatenate(
        [att[(y // WIN) * NW + j][(y % WIN) * WIN:(y % WIN + 1) * WIN]
         for y in range(H) for j in range(NW)], axis=0)
    ssum = a_img + lepe
    o_ref[0] = jnp.dot(ssum, w_ref[...],
                       preferred_element_type=jnp.float32) + bo_ref[...]


def _fuse_call(r_idx, q6, k6, v6, taps, lb, Wop16, bo):
    kv_blk = (1, NH, WIN, NW, WIN, CP)

    def kv_map(b, rr):
        return (b, 0, 0, 0, 0, 0)

    grid_spec = pltpu.PrefetchScalarGridSpec(
        num_scalar_prefetch=1,
        grid=(B,),
        in_specs=[
            pl.BlockSpec(kv_blk, kv_map),
            pl.BlockSpec(kv_blk, kv_map),
            pl.BlockSpec(kv_blk, kv_map),
            pl.BlockSpec((KS * KS, HEADS, HDP), lambda b, rr: (0, 0, 0)),
            pl.BlockSpec((1, HEADS, HDP), lambda b, rr: (0, 0, 0)),
            pl.BlockSpec((CP, C), lambda b, rr: (0, 0)),
            pl.BlockSpec((1, C), lambda b, rr: (0, 0)),
        ],
        out_specs=pl.BlockSpec((1, H * W, C), lambda b, rr: (b, 0, 0)),
        scratch_shapes=[pltpu.VMEM((_FP, HEADS, HDP), jnp.bfloat16)],
    )
    return pl.pallas_call(
        _fuse_body,
        grid_spec=grid_spec,
        out_shape=jax.ShapeDtypeStruct((B, H * W, C), jnp.float32),
    )(r_idx, q6, k6, v6, taps, lb, Wop16, bo)


def _pad_heads(t):
    lead = t.shape[:-1]
    n = t.shape[-1] // HD
    t = t.reshape(lead + (n, HD))
    t = jnp.pad(t, [(0, 0)] * len(lead) + [(0, 0), (0, HDP - HD)])
    return t.reshape(lead + (n * HDP,))


@jax.jit
def _run(x, Wqkv, bqkv, Wo, bo, lepe_w, lepe_b):
    Wp16 = _pad_heads(Wqkv).astype(jnp.bfloat16)
    bp = _pad_heads(bqkv).reshape(1, 3 * CP)
    Wop16 = (jnp.pad(Wo.reshape(HEADS, HD, C),
                     ((0, 0), (0, HDP - HD), (0, 0)))
             .reshape(CP, C).astype(jnp.bfloat16))
    taps = _pad_heads(lepe_w.reshape(C, KS * KS).T).reshape(KS * KS, HEADS, HDP)
    lb = _pad_heads(lepe_b).reshape(1, HEADS, HDP)

    q, k, v, r_idx = _qkv_call(x, Wp16, bp)
    shp6 = (B, NH, WIN, NW, WIN, CP)
    out = _fuse_call(r_idx, q.reshape(shp6), k.reshape(shp6),
                     v.reshape(shp6), taps, lb, Wop16, bo.reshape(1, C))
    return out


def kernel(x, x_size, Wqkv, bqkv, Wo, bo, lepe_w, lepe_b):
    return _run(x, Wqkv, bqkv, Wo, bo, lepe_w, lepe_b)
